# baseline (speedup 1.0000x reference)
import functools

import jax
import jax.numpy as jnp
from jax import lax
from jax.experimental import pallas as pl
from jax.experimental.pallas import tpu as pltpu
from jax.experimental.pallas import tpu_sc as plsc

VOCAB = 100000
D = 128
HID = 2048
B = 4096
L = 50
NHALF = 1
HB = B // NHALF
NVREG = D // 16


def _make_pool_kernel(ng):
    info = plsc.get_sparse_core_info()
    nc, ns = info.num_cores, info.num_subcores
    nw = nc * ns
    gpw = ng // nw
    G = 4
    nchunk = gpw // G
    npair = nchunk // 2
    RU = 1

    mesh = plsc.VectorSubcoreMesh(core_axis_name="c", subcore_axis_name="s")

    @functools.partial(
        pl.kernel,
        mesh=mesh,
        out_type=jax.ShapeDtypeStruct((ng, D), jnp.float32),
        scratch_types=[
            pltpu.VMEM((G, L), jnp.int32),
            pltpu.VMEM((G, L), jnp.int32),
            pltpu.VMEM((G, L, D), jnp.float32),
            pltpu.VMEM((G, L, D), jnp.float32),
            pltpu.VMEM((G, D), jnp.float32),
            pltpu.SemaphoreType.DMA,
            pltpu.SemaphoreType.DMA,
        ],
    )
    def pool(emb_hbm, idx_hbm, out_hbm, idx0, idx1, rows0, rows1, out_v,
             sem0, sem1):
        w = lax.axis_index("s") * nc + lax.axis_index("c")
        w0 = w * gpw

        def fire(c, idx_v, rows_v, sem):
            pltpu.sync_copy(idx_hbm.at[pl.ds(w0 + c * G, G)], idx_v)
            for g in range(G):
                pltpu.async_copy(emb_hbm.at[idx_v.at[g]], rows_v.at[g], sem)

        def drain_acc_store(c, idx_v, rows_v, sem):
            for g in range(G):
                pltpu.make_async_copy(
                    emb_hbm.at[idx_v.at[g]], rows_v.at[g], sem).wait()
            for g in range(G):
                def row_body(r, accs):
                    accs = list(accs)
                    for rr in range(RU):
                        row = r * RU + rr
                        for v in range(NVREG):
                            accs[v] = accs[v] + rows_v[g, row,
                                                       pl.ds(v * 16, 16)]
                    return tuple(accs)
                accs = lax.fori_loop(
                    0, L // RU, row_body,
                    tuple(jnp.zeros((16,), jnp.float32)
                          for _ in range(NVREG)),
                )
                for v in range(NVREG):
                    out_v[g, pl.ds(16 * v, 16)] = accs[v] * (1.0 / L)
            pltpu.sync_copy(out_v, out_hbm.at[pl.ds(w0 + c * G, G)])

        fire(0, idx0, rows0, sem0)

        def pair_body(p, carry):
            c0 = 2 * p
            fire(c0 + 1, idx1, rows1, sem1)
            drain_acc_store(c0, idx0, rows0, sem0)
            fire(c0 + 2, idx0, rows0, sem0)
            drain_acc_store(c0 + 1, idx1, rows1, sem1)
            return carry

        lax.fori_loop(0, npair - 1, pair_body, 0)
        fire(nchunk - 1, idx1, rows1, sem1)
        drain_acc_store(nchunk - 2, idx0, rows0, sem0)
        drain_acc_store(nchunk - 1, idx1, rows1, sem1)

    return pool


BB = 512


def _mlp_body(a_ref, s_ref, t_ref, d_ref, w1_ref, b1_ref, w2_ref, b2_ref,
              out_ref):
    x = jnp.concatenate([a_ref[...], s_ref[...], t_ref[...]], axis=1)
    h = jnp.dot(x, w1_ref[...], preferred_element_type=jnp.float32)
    h = jnp.maximum(h + b1_ref[...], 0.0)
    y = jnp.dot(h, w2_ref[...], preferred_element_type=jnp.float32)
    r = y + b2_ref[...] - d_ref[...]
    out_ref[...] = jnp.mean(r * r, axis=1).reshape(1, BB)


def _mlp(pooled, w1, b1, w2, b2):
    bpt = pooled.shape[0] // 4
    nb = bpt // BB

    def tensor_spec(k):
        return pl.BlockSpec((BB, D), lambda i, k=k: (k * nb + i, 0))

    full = lambda shape: pl.BlockSpec(shape, lambda i: (0,) * len(shape))
    out = pl.pallas_call(
        _mlp_body,
        grid=(nb,),
        in_specs=[
            tensor_spec(0), tensor_spec(1), tensor_spec(2), tensor_spec(3),
            full((3 * D, HID)),
            full((1, HID)),
            full((HID, D)),
            full((1, D)),
        ],
        out_specs=pl.BlockSpec((1, BB), lambda i: (0, i)),
        out_shape=jax.ShapeDtypeStruct((1, bpt), jnp.float32),
    )(pooled, pooled, pooled, pooled, w1, b1.reshape(1, HID), w2,
      b2.reshape(1, D))
    return out.reshape(bpt)


_pool_kernel = None


def kernel(api, seq, token, desc, emb, W1, b1, W2, b2):
    global _pool_kernel
    if _pool_kernel is None:
        _pool_kernel = _make_pool_kernel(4 * HB)
    tensors = [x.astype(jnp.int32) for x in (api, seq, token, desc)]
    outs = []
    for h in range(NHALF):
        sl = slice(h * HB, (h + 1) * HB)
        idx_h = jnp.concatenate([x[sl] for x in tensors])
        pooled = _pool_kernel(emb, idx_h)
        outs.append(_mlp(pooled, W1, b1, W2, b2))
    return jnp.concatenate(outs)


# speedup vs baseline: 1.1611x; 1.1611x over previous
import functools

import jax
import jax.numpy as jnp
from jax import lax
from jax.experimental import pallas as pl
from jax.experimental.pallas import tpu as pltpu
from jax.experimental.pallas import tpu_sc as plsc

VOCAB = 100000
D = 128
HID = 2048
B = 4096
L = 50
NHALF = 1
HB = B // NHALF
NVREG = D // 16


def _make_pool_kernel(ng):
    info = plsc.get_sparse_core_info()
    nc, ns = info.num_cores, info.num_subcores
    nw = nc * ns
    gpw = ng // nw
    G = 8
    nchunk = gpw // G
    npair = nchunk // 2
    RU = 1

    mesh = plsc.VectorSubcoreMesh(core_axis_name="c", subcore_axis_name="s")

    @functools.partial(
        pl.kernel,
        mesh=mesh,
        out_type=jax.ShapeDtypeStruct((ng, D), jnp.float32),
        scratch_types=[
            pltpu.VMEM((G, L), jnp.int32),
            pltpu.VMEM((G, L), jnp.int32),
            pltpu.VMEM((G, L, D), jnp.float32),
            pltpu.VMEM((G, L, D), jnp.float32),
            pltpu.VMEM((G, D), jnp.float32),
            pltpu.SemaphoreType.DMA,
            pltpu.SemaphoreType.DMA,
        ],
    )
    def pool(emb_hbm, idx_hbm, out_hbm, idx0, idx1, rows0, rows1, out_v,
             sem0, sem1):
        w = lax.axis_index("s") * nc + lax.axis_index("c")
        w0 = w * gpw

        def fire(c, idx_v, rows_v, sem):
            pltpu.sync_copy(idx_hbm.at[pl.ds(w0 + c * G, G)], idx_v)
            for g in range(G):
                pltpu.async_copy(emb_hbm.at[idx_v.at[g]], rows_v.at[g], sem)

        def drain_acc_store(c, idx_v, rows_v, sem):
            for g in range(G):
                pltpu.make_async_copy(
                    emb_hbm.at[idx_v.at[g]], rows_v.at[g], sem).wait()
            for g in range(G):
                def row_body(r, accs):
                    accs = list(accs)
                    for rr in range(RU):
                        row = r * RU + rr
                        for v in range(NVREG):
                            accs[v] = accs[v] + rows_v[g, row,
                                                       pl.ds(v * 16, 16)]
                    return tuple(accs)
                accs = lax.fori_loop(
                    0, L // RU, row_body,
                    tuple(jnp.zeros((16,), jnp.float32)
                          for _ in range(NVREG)),
                )
                for v in range(NVREG):
                    out_v[g, pl.ds(16 * v, 16)] = accs[v] * (1.0 / L)
            pltpu.sync_copy(out_v, out_hbm.at[pl.ds(w0 + c * G, G)])

        fire(0, idx0, rows0, sem0)

        def pair_body(p, carry):
            c0 = 2 * p
            fire(c0 + 1, idx1, rows1, sem1)
            drain_acc_store(c0, idx0, rows0, sem0)
            fire(c0 + 2, idx0, rows0, sem0)
            drain_acc_store(c0 + 1, idx1, rows1, sem1)
            return carry

        lax.fori_loop(0, npair - 1, pair_body, 0)
        fire(nchunk - 1, idx1, rows1, sem1)
        drain_acc_store(nchunk - 2, idx0, rows0, sem0)
        drain_acc_store(nchunk - 1, idx1, rows1, sem1)

    return pool


BB = 512


def _mlp_body(a_ref, s_ref, t_ref, d_ref, w1_ref, b1_ref, w2_ref, b2_ref,
              out_ref):
    x = jnp.concatenate([a_ref[...], s_ref[...], t_ref[...]], axis=1)
    h = jnp.dot(x, w1_ref[...], preferred_element_type=jnp.float32)
    h = jnp.maximum(h + b1_ref[...], 0.0)
    y = jnp.dot(h, w2_ref[...], preferred_element_type=jnp.float32)
    r = y + b2_ref[...] - d_ref[...]
    out_ref[...] = jnp.mean(r * r, axis=1).reshape(1, BB)


def _mlp(pooled, w1, b1, w2, b2):
    bpt = pooled.shape[0] // 4
    nb = bpt // BB

    def tensor_spec(k):
        return pl.BlockSpec((BB, D), lambda i, k=k: (k * nb + i, 0))

    full = lambda shape: pl.BlockSpec(shape, lambda i: (0,) * len(shape))
    out = pl.pallas_call(
        _mlp_body,
        grid=(nb,),
        in_specs=[
            tensor_spec(0), tensor_spec(1), tensor_spec(2), tensor_spec(3),
            full((3 * D, HID)),
            full((1, HID)),
            full((HID, D)),
            full((1, D)),
        ],
        out_specs=pl.BlockSpec((1, BB), lambda i: (0, i)),
        out_shape=jax.ShapeDtypeStruct((1, bpt), jnp.float32),
    )(pooled, pooled, pooled, pooled, w1, b1.reshape(1, HID), w2,
      b2.reshape(1, D))
    return out.reshape(bpt)


_pool_kernel = None


def kernel(api, seq, token, desc, emb, W1, b1, W2, b2):
    global _pool_kernel
    if _pool_kernel is None:
        _pool_kernel = _make_pool_kernel(4 * HB)
    tensors = [x.astype(jnp.int32) for x in (api, seq, token, desc)]
    outs = []
    for h in range(NHALF):
        sl = slice(h * HB, (h + 1) * HB)
        idx_h = jnp.concatenate([x[sl] for x in tensors])
        pooled = _pool_kernel(emb, idx_h)
        outs.append(_mlp(pooled, W1, b1, W2, b2))
    return jnp.concatenate(outs)
